# baseline (speedup 1.0000x reference)
import functools
import jax
import jax.numpy as jnp
from jax import lax
from jax.experimental import pallas as pl
from jax.experimental.pallas import tpu as pltpu
from jax.experimental.pallas import tpu_sc as plsc

_B = 16384
_D = 64
_B_SC = 8192
_TC_ROWS = 512


def _make_sc_gather(b_sc):
    info = plsc.get_sparse_core_info()
    nc, ns = info.num_cores, info.num_subcores
    nw = nc * ns
    b_per_w = b_sc // nw
    mesh = plsc.VectorSubcoreMesh(core_axis_name="c", subcore_axis_name="s")

    @functools.partial(
        pl.kernel,
        mesh=mesh,
        out_type=jax.ShapeDtypeStruct((b_sc, _D), jnp.float32),
        scratch_types=[
            pltpu.VMEM((b_per_w,), jnp.int32),
            pltpu.VMEM((b_per_w, _D), jnp.float32),
            pltpu.SemaphoreType.DMA,
        ],
        compiler_params=pltpu.CompilerParams(use_tc_tiling_on_sc=True),
    )
    def sc_gather(idx_hbm, table_hbm, out_hbm, idx_v, rows_v, sem):
        wid = lax.axis_index("s") * nc + lax.axis_index("c")
        base = wid * b_per_w
        pltpu.sync_copy(idx_hbm.at[pl.ds(base, b_per_w)], idx_v)

        def body(c, carry):
            v = idx_v[pl.ds(c * 16, 16)]
            for j in range(16):
                t = v[j]
                pltpu.async_copy(
                    table_hbm.at[pl.ds(t, 1)],
                    rows_v.at[pl.ds(c * 16 + j, 1)],
                    sem,
                )
            return carry

        lax.fori_loop(0, b_per_w // 16, body, 0)

        def drain(i, carry):
            pltpu.make_async_copy(
                table_hbm.at[pl.ds(0, 1)], rows_v.at[pl.ds(i, 1)], sem
            ).wait()
            return carry

        lax.fori_loop(0, b_per_w, drain, 0)
        pltpu.sync_copy(rows_v, out_hbm.at[pl.ds(base, b_per_w)])

    return sc_gather


def _tc_gather_kernel(idx_ref, table_ref, o_ref, sem):
    i = pl.program_id(0)

    def issue(r, carry):
        t = idx_ref[i * _TC_ROWS + r]
        pltpu.make_async_copy(
            table_ref.at[pl.ds(t, 1)], o_ref.at[pl.ds(r, 1)], sem
        ).start()
        return carry

    lax.fori_loop(0, _TC_ROWS, issue, 0)

    def drain(r, carry):
        pltpu.make_async_copy(
            table_ref.at[pl.ds(0, 1)], o_ref.at[pl.ds(r, 1)], sem
        ).wait()
        return carry

    lax.fori_loop(0, _TC_ROWS, drain, 0)


def _tc_gather(idx, table, b_tc):
    grid_spec = pltpu.PrefetchScalarGridSpec(
        num_scalar_prefetch=1,
        grid=(b_tc // _TC_ROWS,),
        in_specs=[pl.BlockSpec(memory_space=pl.ANY)],
        out_specs=pl.BlockSpec((_TC_ROWS, _D), lambda i, idx_ref: (i, 0)),
        scratch_shapes=[pltpu.SemaphoreType.DMA],
    )
    return pl.pallas_call(
        _tc_gather_kernel,
        grid_spec=grid_spec,
        out_shape=jax.ShapeDtypeStruct((b_tc, _D), jnp.float32),
    )(idx, table)


def kernel(g, h, r, norm, table):
    idx = jnp.squeeze(h).astype(jnp.int32)
    out_sc = _make_sc_gather(_B_SC)(idx[:_B_SC], table)
    out_tc = _tc_gather(idx[_B_SC:], table, _B - _B_SC)
    return jnp.concatenate([out_sc, out_tc], axis=0)


# speedup vs baseline: 1.0575x; 1.0575x over previous
import functools
import jax
import jax.numpy as jnp
from jax import lax
from jax.experimental import pallas as pl
from jax.experimental.pallas import tpu as pltpu
from jax.experimental.pallas import tpu_sc as plsc

_B = 16384
_D = 64
_B_SC = 10240
_TC_ROWS = 512


def _make_sc_gather(b_sc):
    info = plsc.get_sparse_core_info()
    nc, ns = info.num_cores, info.num_subcores
    nw = nc * ns
    b_per_w = b_sc // nw
    mesh = plsc.VectorSubcoreMesh(core_axis_name="c", subcore_axis_name="s")

    @functools.partial(
        pl.kernel,
        mesh=mesh,
        out_type=jax.ShapeDtypeStruct((b_sc, _D), jnp.float32),
        scratch_types=[
            pltpu.VMEM((b_per_w,), jnp.int32),
            pltpu.VMEM((b_per_w, _D), jnp.float32),
            pltpu.SemaphoreType.DMA,
        ],
        compiler_params=pltpu.CompilerParams(use_tc_tiling_on_sc=True),
    )
    def sc_gather(idx_hbm, table_hbm, out_hbm, idx_v, rows_v, sem):
        wid = lax.axis_index("s") * nc + lax.axis_index("c")
        base = wid * b_per_w
        pltpu.sync_copy(idx_hbm.at[pl.ds(base, b_per_w)], idx_v)

        def body(c, carry):
            v = idx_v[pl.ds(c * 16, 16)]
            for j in range(16):
                t = v[j]
                pltpu.async_copy(
                    table_hbm.at[pl.ds(t, 1)],
                    rows_v.at[pl.ds(c * 16 + j, 1)],
                    sem,
                )
            return carry

        lax.fori_loop(0, b_per_w // 16, body, 0)

        def drain(i, carry):
            pltpu.make_async_copy(
                table_hbm.at[pl.ds(0, 1)], rows_v.at[pl.ds(i, 1)], sem
            ).wait()
            return carry

        lax.fori_loop(0, b_per_w, drain, 0)
        pltpu.sync_copy(rows_v, out_hbm.at[pl.ds(base, b_per_w)])

    return sc_gather


def _tc_gather_kernel(idx_ref, table_ref, o_ref, sem):
    i = pl.program_id(0)

    def issue(r, carry):
        t = idx_ref[i * _TC_ROWS + r]
        pltpu.make_async_copy(
            table_ref.at[pl.ds(t, 1)], o_ref.at[pl.ds(r, 1)], sem
        ).start()
        return carry

    lax.fori_loop(0, _TC_ROWS, issue, 0)

    def drain(r, carry):
        pltpu.make_async_copy(
            table_ref.at[pl.ds(0, 1)], o_ref.at[pl.ds(r, 1)], sem
        ).wait()
        return carry

    lax.fori_loop(0, _TC_ROWS, drain, 0)


def _tc_gather(idx, table, b_tc):
    grid_spec = pltpu.PrefetchScalarGridSpec(
        num_scalar_prefetch=1,
        grid=(b_tc // _TC_ROWS,),
        in_specs=[pl.BlockSpec(memory_space=pl.ANY)],
        out_specs=pl.BlockSpec((_TC_ROWS, _D), lambda i, idx_ref: (i, 0)),
        scratch_shapes=[pltpu.SemaphoreType.DMA],
    )
    return pl.pallas_call(
        _tc_gather_kernel,
        grid_spec=grid_spec,
        out_shape=jax.ShapeDtypeStruct((b_tc, _D), jnp.float32),
    )(idx, table)


def kernel(g, h, r, norm, table):
    idx = jnp.squeeze(h).astype(jnp.int32)
    out_sc = _make_sc_gather(_B_SC)(idx[:_B_SC], table)
    out_tc = _tc_gather(idx[_B_SC:], table, _B - _B_SC)
    return jnp.concatenate([out_sc, out_tc], axis=0)


# speedup vs baseline: 1.2994x; 1.2288x over previous
import functools
import jax
import jax.numpy as jnp
from jax import lax
from jax.experimental import pallas as pl
from jax.experimental.pallas import tpu as pltpu
from jax.experimental.pallas import tpu_sc as plsc

_B = 16384
_D = 64


def _make_gather(num_nodes):
    info = plsc.get_sparse_core_info()
    nc, ns = info.num_cores, info.num_subcores
    nw = nc * ns
    b_per_w = _B // nw
    mesh = plsc.VectorSubcoreMesh(core_axis_name="c", subcore_axis_name="s")

    @functools.partial(
        pl.kernel,
        mesh=mesh,
        out_type=jax.ShapeDtypeStruct((_B, _D), jnp.float32),
        scratch_types=[
            pltpu.VMEM((b_per_w,), jnp.int32),
            pltpu.VMEM((b_per_w, _D), jnp.float32),
            pltpu.SemaphoreType.DMA,
        ],
        compiler_params=pltpu.CompilerParams(use_tc_tiling_on_sc=True),
    )
    def gather_kernel(idx_hbm, table_hbm, out_hbm, idx_v, rows_v, sem):
        wid = lax.axis_index("s") * nc + lax.axis_index("c")
        base = wid * b_per_w
        pltpu.sync_copy(idx_hbm.at[pl.ds(base, b_per_w)], idx_v)

        def body(c, carry):
            v = idx_v[pl.ds(c * 16, 16)]
            for j in range(16):
                t = v[j]
                pltpu.async_copy(
                    table_hbm.at[pl.ds(t, 1)],
                    rows_v.at[pl.ds(c * 16 + j, 1)],
                    sem,
                )
            return carry

        lax.fori_loop(0, b_per_w // 16, body, 0)

        pltpu.make_async_copy(
            table_hbm.at[pl.ds(0, b_per_w)], rows_v, sem
        ).wait()
        pltpu.sync_copy(rows_v, out_hbm.at[pl.ds(base, b_per_w)])

    return gather_kernel


def kernel(g, h, r, norm, table):
    idx = jnp.squeeze(h).astype(jnp.int32)
    return _make_gather(table.shape[0])(idx, table)
